# baseline (speedup 1.0000x reference)
import functools

import jax
import jax.numpy as jnp
from jax import lax
from jax.experimental import pallas as pl
from jax.experimental.pallas import tpu as pltpu
from jax.experimental.pallas import tpu_sc as plsc

_NC = 2
_NS = 16
_NW = _NC * _NS
_C = 80
_SLABS = 10
_ZR = 40


def _sc_mesh():
    return plsc.VectorSubcoreMesh(
        core_axis_name="c", subcore_axis_name="s",
        num_cores=_NC, num_subcores=_NS)


def _sc_degrees(src3, dst3, ones_hbm, zeros_hbm, n):
    _, nch, c = src3.shape
    slab = n // _SLABS

    @functools.partial(
        pl.kernel,
        out_type=jax.ShapeDtypeStruct((_NC, 2, n, 16), jnp.float32),
        mesh=_sc_mesh(),
        scratch_types=[
            pltpu.VMEM((nch, c), jnp.int32),
            pltpu.VMEM((nch, c), jnp.int32),
            pltpu.VMEM((c, 16), jnp.float32),
            pltpu.VMEM((_ZR, 16), jnp.float32),
            pltpu.VMEM_SHARED((n, 16), jnp.float32),
            pltpu.VMEM_SHARED((n, 16), jnp.float32),
        ],
        compiler_params=pltpu.CompilerParams(use_tc_tiling_on_sc=False),
    )
    def k(src_hbm, dst_hbm, ones_ref, zeros_ref, out_hbm,
          src_v, dst_v, ones_v, zb_v, dout_sh, din_sh):
        cid = lax.axis_index("c")
        sid = lax.axis_index("s")
        wid = sid * _NC + cid
        pltpu.sync_copy(ones_ref, ones_v)
        pltpu.sync_copy(zeros_ref, zb_v)

        @pl.when(sid < _SLABS)
        def _():
            for t in range(slab // _ZR):
                zsl = pl.ds(sid * slab + t * _ZR, _ZR)
                pltpu.sync_copy(zb_v, dout_sh.at[zsl])
                pltpu.sync_copy(zb_v, din_sh.at[zsl])

        pltpu.sync_copy(src_hbm.at[wid], src_v)
        pltpu.sync_copy(dst_hbm.at[wid], dst_v)
        plsc.subcore_barrier()

        def body(j, carry):
            pltpu.sync_copy(ones_v, dout_sh.at[src_v.at[j]], add=True)
            pltpu.sync_copy(ones_v, din_sh.at[dst_v.at[j]], add=True)
            return carry

        lax.fori_loop(0, nch, body, 0)
        plsc.subcore_barrier()

        @pl.when(sid < _SLABS)
        def _():
            sl = pl.ds(sid * slab, slab)
            for t in range(slab // _ZR):
                zsl = pl.ds(sid * slab + t * _ZR, _ZR)
                pltpu.sync_copy(dout_sh.at[zsl], zb_v)
                pltpu.sync_copy(zb_v, out_hbm.at[cid, 0, zsl])
                pltpu.sync_copy(din_sh.at[zsl], zb_v)
                pltpu.sync_copy(zb_v, out_hbm.at[cid, 1, zsl])

    return k(src3, dst3, ones_hbm, zeros_hbm)


def _sc_msg(hn, src2, dst2, zeros_hbm):
    n, d = hn.shape
    _, nch, c = src2.shape
    half = n // _NC
    garbage = half
    nslab = 5
    slab = half // nslab
    nbuf = 10

    @functools.partial(
        pl.kernel,
        out_type=jax.ShapeDtypeStruct((n, d), jnp.float32),
        mesh=_sc_mesh(),
        scratch_types=[
            pltpu.VMEM((nch, c), jnp.int32),
            pltpu.VMEM((nch, c), jnp.int32),
            pltpu.VMEM((nch, c), jnp.int32),
        ] + [pltpu.VMEM((c, d), jnp.float32)] * nbuf + [
            pltpu.VMEM((_ZR, d), jnp.float32),
            pltpu.VMEM_SHARED((half + 8, d), jnp.float32),
        ] + [pltpu.SemaphoreType.DMA] * (2 * nbuf),
        compiler_params=pltpu.CompilerParams(use_tc_tiling_on_sc=False),
    )
    def k(hn_hbm, src_hbm, dst_hbm, z_hbm, out_hbm,
          src_v, dst_v, dstm_v, *rest):
        bufs = rest[:nbuf]
        zb_v = rest[nbuf]
        acc_sh = rest[nbuf + 1]
        gsem = rest[nbuf + 2:2 * nbuf + 2]
        ssem = rest[2 * nbuf + 2:]
        cid = lax.axis_index("c")
        sid = lax.axis_index("s")
        lo = cid * half
        pltpu.sync_copy(z_hbm, zb_v)
        pltpu.sync_copy(src_hbm.at[sid], src_v)
        pltpu.sync_copy(dst_hbm.at[sid], dst_v)

        @pl.when(sid < nslab)
        def _():
            for t in range(slab // _ZR):
                pltpu.sync_copy(
                    zb_v, acc_sh.at[pl.ds(sid * slab + t * _ZR, _ZR)])

        def remap(r, carry):
            for cc in range(c // 16):
                dv = dst_v[r, pl.ds(cc * 16, 16)] - lo
                ok = (dv >= 0) & (dv < half)
                dstm_v[r, pl.ds(cc * 16, 16)] = jnp.where(ok, dv, garbage)
            return carry

        lax.fori_loop(0, nch, remap, 0)
        plsc.subcore_barrier()

        for b in range(nbuf):
            pltpu.async_copy(hn_hbm.at[src_v.at[b]], bufs[b], gsem[b])
        niter = nch // nbuf

        def body(i, carry):
            j = nbuf * i
            for b in range(nbuf):
                pltpu.make_async_copy(
                    hn_hbm.at[src_v.at[0]], bufs[b], gsem[b]).wait()
                pltpu.async_copy(bufs[b], acc_sh.at[dstm_v.at[j + b]],
                                 ssem[b], add=True)

            @pl.when(i < niter - 1)
            def _():
                for b in range(nbuf):
                    pltpu.make_async_copy(
                        bufs[b], acc_sh.at[dstm_v.at[0]], ssem[b]).wait()
                    pltpu.async_copy(hn_hbm.at[src_v.at[j + nbuf + b]],
                                     bufs[b], gsem[b])

            return carry

        lax.fori_loop(0, niter, body, 0)
        for b in range(nbuf):
            pltpu.make_async_copy(bufs[b], acc_sh.at[dstm_v.at[0]],
                                  ssem[b]).wait()
        plsc.subcore_barrier()

        @pl.when(sid < nslab)
        def _():
            for t in range(slab // _ZR):
                sl0 = sid * slab + t * _ZR
                pltpu.sync_copy(acc_sh.at[pl.ds(sl0, _ZR)], zb_v)
                pltpu.sync_copy(zb_v, out_hbm.at[pl.ds(cid * half + sl0, _ZR)])

    return k(hn, src2, dst2, zeros_hbm)


def _tc_merge(x, w, b, deg, r=1000):
    n, d = x.shape
    h = w.shape[0]

    def body(x_ref, w_ref, b_ref, deg_ref, hn_ref, sc_ref):
        dout_p = deg_ref[0, 0, :, 0:1] + deg_ref[1, 0, :, 0:1]
        din_p = deg_ref[0, 1, :, 0:1] + deg_ref[1, 1, :, 0:1]
        dout_s = lax.rsqrt(jnp.maximum(dout_p, 1.0))
        din_s = lax.rsqrt(jnp.maximum(din_p, 1.0))
        h0 = lax.dot_general(x_ref[...], w_ref[...], (((1,), (1,)), ((), ())),
                             preferred_element_type=jnp.float32)
        hn_ref[...] = (h0 + b_ref[...][None, :]) * dout_s
        sc_ref[0] = jnp.broadcast_to(dout_s, (r, 16))
        sc_ref[1] = jnp.broadcast_to(din_s, (r, 16))

    return pl.pallas_call(
        body,
        grid=(n // r,),
        in_specs=[pl.BlockSpec((r, d), lambda i: (i, 0)),
                  pl.BlockSpec((h, d), lambda i: (0, 0)),
                  pl.BlockSpec((h,), lambda i: (0,)),
                  pl.BlockSpec((2, 2, r, 16), lambda i: (0, 0, i, 0))],
        out_specs=[pl.BlockSpec((r, h), lambda i: (i, 0)),
                   pl.BlockSpec((2, r, 16), lambda i: (0, i, 0))],
        out_shape=[jax.ShapeDtypeStruct((n, h), jnp.float32),
                   jax.ShapeDtypeStruct((2, n, 16), jnp.float32)],
    )(x, w, b, deg)


def _tc_layer(agg, scales, w, b, r=1000):
    n, hdim = agg.shape

    def body(a_ref, s_ref, w_ref, b_ref, h_ref, hn_ref):
        dout_s = s_ref[0, :, 0:1]
        din_s = s_ref[1, :, 0:1]
        z = lax.dot_general(a_ref[...] * din_s, w_ref[...],
                            (((1,), (0,)), ((), ())),
                            preferred_element_type=jnp.float32)
        hl = jnp.maximum(z + b_ref[...][None, :], 0.0)
        h_ref[...] = hl
        hn_ref[...] = hl * dout_s

    return pl.pallas_call(
        body,
        grid=(n // r,),
        in_specs=[pl.BlockSpec((r, hdim), lambda i: (i, 0)),
                  pl.BlockSpec((2, r, 16), lambda i: (0, i, 0)),
                  pl.BlockSpec((hdim, hdim), lambda i: (0, 0)),
                  pl.BlockSpec((hdim,), lambda i: (0,))],
        out_specs=[pl.BlockSpec((r, hdim), lambda i: (i, 0)),
                   pl.BlockSpec((r, hdim), lambda i: (i, 0))],
        out_shape=[jax.ShapeDtypeStruct((n, hdim), jnp.float32),
                   jax.ShapeDtypeStruct((n, hdim), jnp.float32)],
    )(agg, scales, w, b)


def _tc_mlp(h1, h2, h3, w0a, w0b, w0c, b0, w1, b1, w2, b2, r=1024):
    n, hdim = h1.shape
    h2w = w0a.shape[0]
    h1w = w1.shape[0]

    def body(h1_ref, h2_ref, h3_ref, w0a_ref, w0b_ref, w0c_ref, b0_ref,
             w1_ref, b1_ref, w2_ref, b2_ref, o_ref):
        dn = (((1,), (1,)), ((), ()))
        z = (lax.dot_general(h1_ref[...], w0a_ref[...], dn,
                             preferred_element_type=jnp.float32)
             + lax.dot_general(h2_ref[...], w0b_ref[...], dn,
                               preferred_element_type=jnp.float32)
             + lax.dot_general(h3_ref[...], w0c_ref[...], dn,
                               preferred_element_type=jnp.float32))
        a0 = jnp.maximum(z + b0_ref[...][None, :], 0.0)
        a1 = lax.dot_general(a0, w1_ref[...], dn,
                             preferred_element_type=jnp.float32)
        a1 = jnp.maximum(a1 + b1_ref[...][None, :], 0.0)
        logit = jnp.sum(a1 * w2_ref[...], axis=1) + b2_ref[0]
        o_ref[...] = 1.0 / (1.0 + jnp.exp(-logit))

    return pl.pallas_call(
        body,
        grid=(pl.cdiv(n, r),),
        in_specs=[pl.BlockSpec((r, hdim), lambda i: (i, 0)),
                  pl.BlockSpec((r, hdim), lambda i: (i, 0)),
                  pl.BlockSpec((r, hdim), lambda i: (i, 0)),
                  pl.BlockSpec((h2w, hdim), lambda i: (0, 0)),
                  pl.BlockSpec((h2w, hdim), lambda i: (0, 0)),
                  pl.BlockSpec((h2w, hdim), lambda i: (0, 0)),
                  pl.BlockSpec((h2w,), lambda i: (0,)),
                  pl.BlockSpec((h1w, h2w), lambda i: (0, 0)),
                  pl.BlockSpec((h1w,), lambda i: (0,)),
                  pl.BlockSpec((1, h1w), lambda i: (0, 0)),
                  pl.BlockSpec((1,), lambda i: (0,))],
        out_specs=pl.BlockSpec((r,), lambda i: (i,)),
        out_shape=jax.ShapeDtypeStruct((n,), jnp.float32),
    )(h1, h2, h3, w0a, w0b, w0c, b0, w1, b1, w2, b2)


def kernel(x, edge_index, edge_types, merge_W, merge_b,
           gcn_W0, gcn_b0, gcn_W1, gcn_b1, gcn_W2, gcn_b2,
           mlp_W0, mlp_b0, mlp_W1, mlp_b1, mlp_W2, mlp_b2):
    n, d = x.shape
    e = edge_index.shape[1]
    nch = e // (_NW * _C)
    src3 = edge_index[0].reshape(_NW, nch, _C)
    dst3 = edge_index[1].reshape(_NW, nch, _C)
    cq = 16
    nch2 = e // (_NS * cq)
    src2 = edge_index[0].reshape(_NS, nch2, cq)
    dst2 = edge_index[1].reshape(_NS, nch2, cq)
    ones16 = jnp.ones((_C, 16), jnp.float32)
    zeros16 = jnp.zeros((_ZR, 16), jnp.float32)
    zeros_rows = jnp.zeros((_ZR, d), jnp.float32)

    deg = _sc_degrees(src3, dst3, ones16, zeros16, n)
    hn, scales = _tc_merge(x, merge_W, merge_b, deg)

    hs = []
    for w, b in ((gcn_W0, gcn_b0), (gcn_W1, gcn_b1), (gcn_W2, gcn_b2)):
        agg = _sc_msg(hn, src2, dst2, zeros_rows)
        h_l, hn = _tc_layer(agg, scales, w, b)
        hs.append(h_l)

    hdim = d
    w0a = mlp_W0[:, :hdim]
    w0b = mlp_W0[:, hdim:2 * hdim]
    w0c = mlp_W0[:, 2 * hdim:]
    return _tc_mlp(hs[0], hs[1], hs[2], w0a, w0b, w0c, mlp_b0,
                   mlp_W1, mlp_b1, mlp_W2, mlp_b2)


# speedup vs baseline: 1.0201x; 1.0201x over previous
import functools

import jax
import jax.numpy as jnp
from jax import lax
from jax.experimental import pallas as pl
from jax.experimental.pallas import tpu as pltpu
from jax.experimental.pallas import tpu_sc as plsc

_NC = 2
_NS = 16
_NW = _NC * _NS
_C = 80
_SLABS = 10
_ZR = 40


def _sc_mesh():
    return plsc.VectorSubcoreMesh(
        core_axis_name="c", subcore_axis_name="s",
        num_cores=_NC, num_subcores=_NS)


def _sc_degrees(src3, dst3, ones_hbm, zeros_hbm, n):
    _, nch, c = src3.shape
    slab = n // _SLABS

    @functools.partial(
        pl.kernel,
        out_type=jax.ShapeDtypeStruct((_NC, 2, n, 16), jnp.float32),
        mesh=_sc_mesh(),
        scratch_types=[
            pltpu.VMEM((nch, c), jnp.int32),
            pltpu.VMEM((nch, c), jnp.int32),
            pltpu.VMEM((c, 16), jnp.float32),
            pltpu.VMEM((_ZR, 16), jnp.float32),
            pltpu.VMEM_SHARED((n, 16), jnp.float32),
            pltpu.VMEM_SHARED((n, 16), jnp.float32),
        ],
        compiler_params=pltpu.CompilerParams(use_tc_tiling_on_sc=False),
    )
    def k(src_hbm, dst_hbm, ones_ref, zeros_ref, out_hbm,
          src_v, dst_v, ones_v, zb_v, dout_sh, din_sh):
        cid = lax.axis_index("c")
        sid = lax.axis_index("s")
        wid = sid * _NC + cid
        pltpu.sync_copy(ones_ref, ones_v)
        pltpu.sync_copy(zeros_ref, zb_v)

        @pl.when(sid < _SLABS)
        def _():
            for t in range(slab // _ZR):
                zsl = pl.ds(sid * slab + t * _ZR, _ZR)
                pltpu.sync_copy(zb_v, dout_sh.at[zsl])
                pltpu.sync_copy(zb_v, din_sh.at[zsl])

        pltpu.sync_copy(src_hbm.at[wid], src_v)
        pltpu.sync_copy(dst_hbm.at[wid], dst_v)
        plsc.subcore_barrier()

        def body(j, carry):
            pltpu.sync_copy(ones_v, dout_sh.at[src_v.at[j]], add=True)
            pltpu.sync_copy(ones_v, din_sh.at[dst_v.at[j]], add=True)
            return carry

        lax.fori_loop(0, nch, body, 0)
        plsc.subcore_barrier()

        @pl.when(sid < _SLABS)
        def _():
            sl = pl.ds(sid * slab, slab)
            for t in range(slab // _ZR):
                zsl = pl.ds(sid * slab + t * _ZR, _ZR)
                pltpu.sync_copy(dout_sh.at[zsl], zb_v)
                pltpu.sync_copy(zb_v, out_hbm.at[cid, 0, zsl])
                pltpu.sync_copy(din_sh.at[zsl], zb_v)
                pltpu.sync_copy(zb_v, out_hbm.at[cid, 1, zsl])

    return k(src3, dst3, ones_hbm, zeros_hbm)


def _sc_msg(hn, src2, dst2, zeros_hbm):
    n, d = hn.shape
    _, nch, c = src2.shape
    half = n // _NC
    garbage = half
    nslab = 5
    slab = half // nslab
    nbuf = 5

    @functools.partial(
        pl.kernel,
        out_type=jax.ShapeDtypeStruct((n, d), jnp.float32),
        mesh=_sc_mesh(),
        scratch_types=[
            pltpu.VMEM((nch, c), jnp.int32),
            pltpu.VMEM((nch, c), jnp.int32),
            pltpu.VMEM((nch, c), jnp.int32),
        ] + [pltpu.VMEM((c, d), jnp.float32)] * nbuf + [
            pltpu.VMEM((_ZR, d), jnp.float32),
            pltpu.VMEM_SHARED((half + 8, d), jnp.float32),
        ] + [pltpu.SemaphoreType.DMA] * (2 * nbuf),
        compiler_params=pltpu.CompilerParams(use_tc_tiling_on_sc=False),
    )
    def k(hn_hbm, src_hbm, dst_hbm, z_hbm, out_hbm,
          src_v, dst_v, dstm_v, *rest):
        bufs = rest[:nbuf]
        zb_v = rest[nbuf]
        acc_sh = rest[nbuf + 1]
        gsem = rest[nbuf + 2:2 * nbuf + 2]
        ssem = rest[2 * nbuf + 2:]
        cid = lax.axis_index("c")
        sid = lax.axis_index("s")
        lo = cid * half
        pltpu.sync_copy(z_hbm, zb_v)
        pltpu.sync_copy(src_hbm.at[sid], src_v)
        pltpu.sync_copy(dst_hbm.at[sid], dst_v)

        @pl.when(sid < nslab)
        def _():
            for t in range(slab // _ZR):
                pltpu.sync_copy(
                    zb_v, acc_sh.at[pl.ds(sid * slab + t * _ZR, _ZR)])

        def remap(r, carry):
            for cc in range(c // 16):
                dv = dst_v[r, pl.ds(cc * 16, 16)] - lo
                ok = (dv >= 0) & (dv < half)
                dstm_v[r, pl.ds(cc * 16, 16)] = jnp.where(ok, dv, garbage)
            return carry

        lax.fori_loop(0, nch, remap, 0)
        plsc.subcore_barrier()

        for b in range(nbuf):
            pltpu.async_copy(hn_hbm.at[src_v.at[b]], bufs[b], gsem[b])
        niter = nch // nbuf

        def body(i, carry):
            j = nbuf * i
            for b in range(nbuf):
                pltpu.make_async_copy(
                    hn_hbm.at[src_v.at[0]], bufs[b], gsem[b]).wait()
                pltpu.async_copy(bufs[b], acc_sh.at[dstm_v.at[j + b]],
                                 ssem[b], add=True)

            @pl.when(i < niter - 1)
            def _():
                for b in range(nbuf):
                    pltpu.make_async_copy(
                        bufs[b], acc_sh.at[dstm_v.at[0]], ssem[b]).wait()
                    pltpu.async_copy(hn_hbm.at[src_v.at[j + nbuf + b]],
                                     bufs[b], gsem[b])

            return carry

        lax.fori_loop(0, niter, body, 0)
        for b in range(nbuf):
            pltpu.make_async_copy(bufs[b], acc_sh.at[dstm_v.at[0]],
                                  ssem[b]).wait()
        plsc.subcore_barrier()

        @pl.when(sid < nslab)
        def _():
            for t in range(slab // _ZR):
                sl0 = sid * slab + t * _ZR
                pltpu.sync_copy(acc_sh.at[pl.ds(sl0, _ZR)], zb_v)
                pltpu.sync_copy(zb_v, out_hbm.at[pl.ds(cid * half + sl0, _ZR)])

    return k(hn, src2, dst2, zeros_hbm)


def _tc_merge(x, w, b, deg, r=1000):
    n, d = x.shape
    h = w.shape[0]

    def body(x_ref, w_ref, b_ref, deg_ref, hn_ref, sc_ref):
        dout_p = deg_ref[0, 0, :, 0:1] + deg_ref[1, 0, :, 0:1]
        din_p = deg_ref[0, 1, :, 0:1] + deg_ref[1, 1, :, 0:1]
        dout_s = lax.rsqrt(jnp.maximum(dout_p, 1.0))
        din_s = lax.rsqrt(jnp.maximum(din_p, 1.0))
        h0 = lax.dot_general(x_ref[...], w_ref[...], (((1,), (1,)), ((), ())),
                             preferred_element_type=jnp.float32)
        hn_ref[...] = (h0 + b_ref[...][None, :]) * dout_s
        sc_ref[0] = jnp.broadcast_to(dout_s, (r, 16))
        sc_ref[1] = jnp.broadcast_to(din_s, (r, 16))

    return pl.pallas_call(
        body,
        grid=(n // r,),
        in_specs=[pl.BlockSpec((r, d), lambda i: (i, 0)),
                  pl.BlockSpec((h, d), lambda i: (0, 0)),
                  pl.BlockSpec((h,), lambda i: (0,)),
                  pl.BlockSpec((2, 2, r, 16), lambda i: (0, 0, i, 0))],
        out_specs=[pl.BlockSpec((r, h), lambda i: (i, 0)),
                   pl.BlockSpec((2, r, 16), lambda i: (0, i, 0))],
        out_shape=[jax.ShapeDtypeStruct((n, h), jnp.float32),
                   jax.ShapeDtypeStruct((2, n, 16), jnp.float32)],
    )(x, w, b, deg)


def _tc_layer(agg, scales, w, b, r=1000):
    n, hdim = agg.shape

    def body(a_ref, s_ref, w_ref, b_ref, h_ref, hn_ref):
        dout_s = s_ref[0, :, 0:1]
        din_s = s_ref[1, :, 0:1]
        z = lax.dot_general(a_ref[...] * din_s, w_ref[...],
                            (((1,), (0,)), ((), ())),
                            preferred_element_type=jnp.float32)
        hl = jnp.maximum(z + b_ref[...][None, :], 0.0)
        h_ref[...] = hl
        hn_ref[...] = hl * dout_s

    return pl.pallas_call(
        body,
        grid=(n // r,),
        in_specs=[pl.BlockSpec((r, hdim), lambda i: (i, 0)),
                  pl.BlockSpec((2, r, 16), lambda i: (0, i, 0)),
                  pl.BlockSpec((hdim, hdim), lambda i: (0, 0)),
                  pl.BlockSpec((hdim,), lambda i: (0,))],
        out_specs=[pl.BlockSpec((r, hdim), lambda i: (i, 0)),
                   pl.BlockSpec((r, hdim), lambda i: (i, 0))],
        out_shape=[jax.ShapeDtypeStruct((n, hdim), jnp.float32),
                   jax.ShapeDtypeStruct((n, hdim), jnp.float32)],
    )(agg, scales, w, b)


def _tc_mlp(h1, h2, h3, w0a, w0b, w0c, b0, w1, b1, w2, b2, r=1024):
    n, hdim = h1.shape
    h2w = w0a.shape[0]
    h1w = w1.shape[0]

    def body(h1_ref, h2_ref, h3_ref, w0a_ref, w0b_ref, w0c_ref, b0_ref,
             w1_ref, b1_ref, w2_ref, b2_ref, o_ref):
        dn = (((1,), (1,)), ((), ()))
        z = (lax.dot_general(h1_ref[...], w0a_ref[...], dn,
                             preferred_element_type=jnp.float32)
             + lax.dot_general(h2_ref[...], w0b_ref[...], dn,
                               preferred_element_type=jnp.float32)
             + lax.dot_general(h3_ref[...], w0c_ref[...], dn,
                               preferred_element_type=jnp.float32))
        a0 = jnp.maximum(z + b0_ref[...][None, :], 0.0)
        a1 = lax.dot_general(a0, w1_ref[...], dn,
                             preferred_element_type=jnp.float32)
        a1 = jnp.maximum(a1 + b1_ref[...][None, :], 0.0)
        logit = jnp.sum(a1 * w2_ref[...], axis=1) + b2_ref[0]
        o_ref[...] = 1.0 / (1.0 + jnp.exp(-logit))

    return pl.pallas_call(
        body,
        grid=(pl.cdiv(n, r),),
        in_specs=[pl.BlockSpec((r, hdim), lambda i: (i, 0)),
                  pl.BlockSpec((r, hdim), lambda i: (i, 0)),
                  pl.BlockSpec((r, hdim), lambda i: (i, 0)),
                  pl.BlockSpec((h2w, hdim), lambda i: (0, 0)),
                  pl.BlockSpec((h2w, hdim), lambda i: (0, 0)),
                  pl.BlockSpec((h2w, hdim), lambda i: (0, 0)),
                  pl.BlockSpec((h2w,), lambda i: (0,)),
                  pl.BlockSpec((h1w, h2w), lambda i: (0, 0)),
                  pl.BlockSpec((h1w,), lambda i: (0,)),
                  pl.BlockSpec((1, h1w), lambda i: (0, 0)),
                  pl.BlockSpec((1,), lambda i: (0,))],
        out_specs=pl.BlockSpec((r,), lambda i: (i,)),
        out_shape=jax.ShapeDtypeStruct((n,), jnp.float32),
    )(h1, h2, h3, w0a, w0b, w0c, b0, w1, b1, w2, b2)


def _tc_layer3_mlp(agg, scales, w, b, h1, h2,
                   w0a, w0b, w0c, b0, w1, b1, w2, b2, r=1024):
    n, hdim = agg.shape
    h2w = w0a.shape[0]
    h1w = w1.shape[0]

    def body(a_ref, s_ref, w_ref, b_ref, h1_ref, h2_ref,
             w0a_ref, w0b_ref, w0c_ref, b0_ref, w1_ref, b1_ref,
             w2_ref, b2_ref, o_ref):
        din_s = s_ref[1, :, 0:1]
        z3 = lax.dot_general(a_ref[...] * din_s, w_ref[...],
                             (((1,), (0,)), ((), ())),
                             preferred_element_type=jnp.float32)
        h3 = jnp.maximum(z3 + b_ref[...][None, :], 0.0)
        dn = (((1,), (1,)), ((), ()))
        z = (lax.dot_general(h1_ref[...], w0a_ref[...], dn,
                             preferred_element_type=jnp.float32)
             + lax.dot_general(h2_ref[...], w0b_ref[...], dn,
                               preferred_element_type=jnp.float32)
             + lax.dot_general(h3, w0c_ref[...], dn,
                               preferred_element_type=jnp.float32))
        a0 = jnp.maximum(z + b0_ref[...][None, :], 0.0)
        a1 = lax.dot_general(a0, w1_ref[...], dn,
                             preferred_element_type=jnp.float32)
        a1 = jnp.maximum(a1 + b1_ref[...][None, :], 0.0)
        logit = jnp.sum(a1 * w2_ref[...], axis=1) + b2_ref[0]
        o_ref[...] = 1.0 / (1.0 + jnp.exp(-logit))

    return pl.pallas_call(
        body,
        grid=(pl.cdiv(n, r),),
        in_specs=[pl.BlockSpec((r, hdim), lambda i: (i, 0)),
                  pl.BlockSpec((2, r, 16), lambda i: (0, i, 0)),
                  pl.BlockSpec((hdim, hdim), lambda i: (0, 0)),
                  pl.BlockSpec((hdim,), lambda i: (0,)),
                  pl.BlockSpec((r, hdim), lambda i: (i, 0)),
                  pl.BlockSpec((r, hdim), lambda i: (i, 0)),
                  pl.BlockSpec((h2w, hdim), lambda i: (0, 0)),
                  pl.BlockSpec((h2w, hdim), lambda i: (0, 0)),
                  pl.BlockSpec((h2w, hdim), lambda i: (0, 0)),
                  pl.BlockSpec((h2w,), lambda i: (0,)),
                  pl.BlockSpec((h1w, h2w), lambda i: (0, 0)),
                  pl.BlockSpec((h1w,), lambda i: (0,)),
                  pl.BlockSpec((1, h1w), lambda i: (0, 0)),
                  pl.BlockSpec((1,), lambda i: (0,))],
        out_specs=pl.BlockSpec((r,), lambda i: (i,)),
        out_shape=jax.ShapeDtypeStruct((n,), jnp.float32),
    )(agg, scales, w, b, h1, h2, w0a, w0b, w0c, b0, w1, b1, w2, b2)


def kernel(x, edge_index, edge_types, merge_W, merge_b,
           gcn_W0, gcn_b0, gcn_W1, gcn_b1, gcn_W2, gcn_b2,
           mlp_W0, mlp_b0, mlp_W1, mlp_b1, mlp_W2, mlp_b2):
    n, d = x.shape
    e = edge_index.shape[1]
    nch = e // (_NW * _C)
    src3 = edge_index[0].reshape(_NW, nch, _C)
    dst3 = edge_index[1].reshape(_NW, nch, _C)
    cq = 32
    nch2 = e // (_NS * cq)
    src2 = edge_index[0].reshape(_NS, nch2, cq)
    dst2 = edge_index[1].reshape(_NS, nch2, cq)
    ones16 = jnp.ones((_C, 16), jnp.float32)
    zeros16 = jnp.zeros((_ZR, 16), jnp.float32)
    zeros_rows = jnp.zeros((_ZR, d), jnp.float32)

    deg = _sc_degrees(src3, dst3, ones16, zeros16, n)
    hn, scales = _tc_merge(x, merge_W, merge_b, deg)

    hs = []
    for w, b in ((gcn_W0, gcn_b0), (gcn_W1, gcn_b1)):
        agg = _sc_msg(hn, src2, dst2, zeros_rows)
        h_l, hn = _tc_layer(agg, scales, w, b)
        hs.append(h_l)
    agg3 = _sc_msg(hn, src2, dst2, zeros_rows)

    hdim = d
    w0a = mlp_W0[:, :hdim]
    w0b = mlp_W0[:, hdim:2 * hdim]
    w0c = mlp_W0[:, 2 * hdim:]
    return _tc_layer3_mlp(agg3, scales, gcn_W2, gcn_b2, hs[0], hs[1],
                          w0a, w0b, w0c, mlp_b0, mlp_W1, mlp_b1,
                          mlp_W2, mlp_b2)


# speedup vs baseline: 1.0352x; 1.0148x over previous
import functools

import jax
import jax.numpy as jnp
from jax import lax
from jax.experimental import pallas as pl
from jax.experimental.pallas import tpu as pltpu
from jax.experimental.pallas import tpu_sc as plsc

_NC = 2
_NS = 16
_NW = _NC * _NS
_C = 80
_SLABS = 10
_ZR = 40


def _sc_mesh():
    return plsc.VectorSubcoreMesh(
        core_axis_name="c", subcore_axis_name="s",
        num_cores=_NC, num_subcores=_NS)


def _sc_degrees(src3, dst3, ones_hbm, zeros_hbm, n):
    _, nch, c = src3.shape
    slab = n // _SLABS

    @functools.partial(
        pl.kernel,
        out_type=jax.ShapeDtypeStruct((_NC, 2, n, 16), jnp.float32),
        mesh=_sc_mesh(),
        scratch_types=[
            pltpu.VMEM((nch, c), jnp.int32),
            pltpu.VMEM((nch, c), jnp.int32),
            pltpu.VMEM((c, 16), jnp.float32),
            pltpu.VMEM((_ZR, 16), jnp.float32),
            pltpu.VMEM_SHARED((n, 16), jnp.float32),
            pltpu.VMEM_SHARED((n, 16), jnp.float32),
            pltpu.SemaphoreType.DMA,
            pltpu.SemaphoreType.DMA,
        ],
        compiler_params=pltpu.CompilerParams(use_tc_tiling_on_sc=False),
    )
    def k(src_hbm, dst_hbm, ones_ref, zeros_ref, out_hbm,
          src_v, dst_v, ones_v, zb_v, dout_sh, din_sh, so, si):
        cid = lax.axis_index("c")
        sid = lax.axis_index("s")
        wid = sid * _NC + cid
        pltpu.sync_copy(ones_ref, ones_v)
        pltpu.sync_copy(zeros_ref, zb_v)

        @pl.when(sid < _SLABS)
        def _():
            for t in range(slab // _ZR):
                zsl = pl.ds(sid * slab + t * _ZR, _ZR)
                pltpu.sync_copy(zb_v, dout_sh.at[zsl])
                pltpu.sync_copy(zb_v, din_sh.at[zsl])

        pltpu.sync_copy(src_hbm.at[wid], src_v)
        pltpu.sync_copy(dst_hbm.at[wid], dst_v)
        plsc.subcore_barrier()

        win = 8

        def body(j, carry):
            pltpu.async_copy(ones_v, dout_sh.at[src_v.at[j]], so, add=True)
            pltpu.async_copy(ones_v, din_sh.at[dst_v.at[j]], si, add=True)

            @pl.when(j >= win)
            def _():
                pltpu.make_async_copy(ones_v, dout_sh.at[src_v.at[0]],
                                      so).wait()
                pltpu.make_async_copy(ones_v, din_sh.at[dst_v.at[0]],
                                      si).wait()

            return carry

        lax.fori_loop(0, nch, body, 0)
        for _ in range(win):
            pltpu.make_async_copy(ones_v, dout_sh.at[src_v.at[0]], so).wait()
            pltpu.make_async_copy(ones_v, din_sh.at[dst_v.at[0]], si).wait()
        plsc.subcore_barrier()

        @pl.when(sid < _SLABS)
        def _():
            sl = pl.ds(sid * slab, slab)
            for t in range(slab // _ZR):
                zsl = pl.ds(sid * slab + t * _ZR, _ZR)
                pltpu.sync_copy(dout_sh.at[zsl], zb_v)
                pltpu.sync_copy(zb_v, out_hbm.at[cid, 0, zsl])
                pltpu.sync_copy(din_sh.at[zsl], zb_v)
                pltpu.sync_copy(zb_v, out_hbm.at[cid, 1, zsl])

    return k(src3, dst3, ones_hbm, zeros_hbm)


def _sc_msg(hn, src2, dst2, zeros_hbm):
    n, d = hn.shape
    _, nch, c = src2.shape
    half = n // _NC
    garbage = half
    nslab = 5
    slab = half // nslab
    nbuf = 5

    @functools.partial(
        pl.kernel,
        out_type=jax.ShapeDtypeStruct((n, d), jnp.float32),
        mesh=_sc_mesh(),
        scratch_types=[
            pltpu.VMEM((nch, c), jnp.int32),
            pltpu.VMEM((nch, c), jnp.int32),
            pltpu.VMEM((nch, c), jnp.int32),
        ] + [pltpu.VMEM((c, d), jnp.float32)] * nbuf + [
            pltpu.VMEM((_ZR, d), jnp.float32),
            pltpu.VMEM_SHARED((half + 8, d), jnp.float32),
        ] + [pltpu.SemaphoreType.DMA] * (2 * nbuf),
        compiler_params=pltpu.CompilerParams(use_tc_tiling_on_sc=False),
    )
    def k(hn_hbm, src_hbm, dst_hbm, z_hbm, out_hbm,
          src_v, dst_v, dstm_v, *rest):
        bufs = rest[:nbuf]
        zb_v = rest[nbuf]
        acc_sh = rest[nbuf + 1]
        gsem = rest[nbuf + 2:2 * nbuf + 2]
        ssem = rest[2 * nbuf + 2:]
        cid = lax.axis_index("c")
        sid = lax.axis_index("s")
        lo = cid * half
        pltpu.sync_copy(z_hbm, zb_v)
        pltpu.sync_copy(src_hbm.at[sid], src_v)
        pltpu.sync_copy(dst_hbm.at[sid], dst_v)

        @pl.when(sid < nslab)
        def _():
            for t in range(slab // _ZR):
                pltpu.sync_copy(
                    zb_v, acc_sh.at[pl.ds(sid * slab + t * _ZR, _ZR)])

        def remap(r, carry):
            for cc in range(c // 16):
                dv = dst_v[r, pl.ds(cc * 16, 16)] - lo
                ok = (dv >= 0) & (dv < half)
                dstm_v[r, pl.ds(cc * 16, 16)] = jnp.where(ok, dv, garbage)
            return carry

        lax.fori_loop(0, nch, remap, 0)
        plsc.subcore_barrier()

        for b in range(nbuf):
            pltpu.async_copy(hn_hbm.at[src_v.at[b]], bufs[b], gsem[b])
        niter = nch // nbuf

        def body(i, carry):
            j = nbuf * i
            for b in range(nbuf):
                pltpu.make_async_copy(
                    hn_hbm.at[src_v.at[0]], bufs[b], gsem[b]).wait()
                pltpu.async_copy(bufs[b], acc_sh.at[dstm_v.at[j + b]],
                                 ssem[b], add=True)

            @pl.when(i < niter - 1)
            def _():
                for b in range(nbuf):
                    pltpu.make_async_copy(
                        bufs[b], acc_sh.at[dstm_v.at[0]], ssem[b]).wait()
                    pltpu.async_copy(hn_hbm.at[src_v.at[j + nbuf + b]],
                                     bufs[b], gsem[b])

            return carry

        lax.fori_loop(0, niter, body, 0)
        for b in range(nbuf):
            pltpu.make_async_copy(bufs[b], acc_sh.at[dstm_v.at[0]],
                                  ssem[b]).wait()
        plsc.subcore_barrier()

        @pl.when(sid < nslab)
        def _():
            for t in range(slab // _ZR):
                sl0 = sid * slab + t * _ZR
                pltpu.sync_copy(acc_sh.at[pl.ds(sl0, _ZR)], zb_v)
                pltpu.sync_copy(zb_v, out_hbm.at[pl.ds(cid * half + sl0, _ZR)])

    return k(hn, src2, dst2, zeros_hbm)


def _tc_merge(x, w, b, deg, r=1000):
    n, d = x.shape
    h = w.shape[0]

    def body(x_ref, w_ref, b_ref, deg_ref, hn_ref, sc_ref):
        dout_p = deg_ref[0, 0, :, 0:1] + deg_ref[1, 0, :, 0:1]
        din_p = deg_ref[0, 1, :, 0:1] + deg_ref[1, 1, :, 0:1]
        dout_s = lax.rsqrt(jnp.maximum(dout_p, 1.0))
        din_s = lax.rsqrt(jnp.maximum(din_p, 1.0))
        h0 = lax.dot_general(x_ref[...], w_ref[...], (((1,), (1,)), ((), ())),
                             preferred_element_type=jnp.float32)
        hn_ref[...] = (h0 + b_ref[...][None, :]) * dout_s
        sc_ref[0] = jnp.broadcast_to(dout_s, (r, 16))
        sc_ref[1] = jnp.broadcast_to(din_s, (r, 16))

    return pl.pallas_call(
        body,
        grid=(n // r,),
        in_specs=[pl.BlockSpec((r, d), lambda i: (i, 0)),
                  pl.BlockSpec((h, d), lambda i: (0, 0)),
                  pl.BlockSpec((h,), lambda i: (0,)),
                  pl.BlockSpec((2, 2, r, 16), lambda i: (0, 0, i, 0))],
        out_specs=[pl.BlockSpec((r, h), lambda i: (i, 0)),
                   pl.BlockSpec((2, r, 16), lambda i: (0, i, 0))],
        out_shape=[jax.ShapeDtypeStruct((n, h), jnp.float32),
                   jax.ShapeDtypeStruct((2, n, 16), jnp.float32)],
    )(x, w, b, deg)


def _tc_layer(agg, scales, w, b, r=1000):
    n, hdim = agg.shape

    def body(a_ref, s_ref, w_ref, b_ref, h_ref, hn_ref):
        dout_s = s_ref[0, :, 0:1]
        din_s = s_ref[1, :, 0:1]
        z = lax.dot_general(a_ref[...] * din_s, w_ref[...],
                            (((1,), (0,)), ((), ())),
                            preferred_element_type=jnp.float32)
        hl = jnp.maximum(z + b_ref[...][None, :], 0.0)
        h_ref[...] = hl
        hn_ref[...] = hl * dout_s

    return pl.pallas_call(
        body,
        grid=(n // r,),
        in_specs=[pl.BlockSpec((r, hdim), lambda i: (i, 0)),
                  pl.BlockSpec((2, r, 16), lambda i: (0, i, 0)),
                  pl.BlockSpec((hdim, hdim), lambda i: (0, 0)),
                  pl.BlockSpec((hdim,), lambda i: (0,))],
        out_specs=[pl.BlockSpec((r, hdim), lambda i: (i, 0)),
                   pl.BlockSpec((r, hdim), lambda i: (i, 0))],
        out_shape=[jax.ShapeDtypeStruct((n, hdim), jnp.float32),
                   jax.ShapeDtypeStruct((n, hdim), jnp.float32)],
    )(agg, scales, w, b)


def _tc_mlp(h1, h2, h3, w0a, w0b, w0c, b0, w1, b1, w2, b2, r=1024):
    n, hdim = h1.shape
    h2w = w0a.shape[0]
    h1w = w1.shape[0]

    def body(h1_ref, h2_ref, h3_ref, w0a_ref, w0b_ref, w0c_ref, b0_ref,
             w1_ref, b1_ref, w2_ref, b2_ref, o_ref):
        dn = (((1,), (1,)), ((), ()))
        z = (lax.dot_general(h1_ref[...], w0a_ref[...], dn,
                             preferred_element_type=jnp.float32)
             + lax.dot_general(h2_ref[...], w0b_ref[...], dn,
                               preferred_element_type=jnp.float32)
             + lax.dot_general(h3_ref[...], w0c_ref[...], dn,
                               preferred_element_type=jnp.float32))
        a0 = jnp.maximum(z + b0_ref[...][None, :], 0.0)
        a1 = lax.dot_general(a0, w1_ref[...], dn,
                             preferred_element_type=jnp.float32)
        a1 = jnp.maximum(a1 + b1_ref[...][None, :], 0.0)
        logit = jnp.sum(a1 * w2_ref[...], axis=1) + b2_ref[0]
        o_ref[...] = 1.0 / (1.0 + jnp.exp(-logit))

    return pl.pallas_call(
        body,
        grid=(pl.cdiv(n, r),),
        in_specs=[pl.BlockSpec((r, hdim), lambda i: (i, 0)),
                  pl.BlockSpec((r, hdim), lambda i: (i, 0)),
                  pl.BlockSpec((r, hdim), lambda i: (i, 0)),
                  pl.BlockSpec((h2w, hdim), lambda i: (0, 0)),
                  pl.BlockSpec((h2w, hdim), lambda i: (0, 0)),
                  pl.BlockSpec((h2w, hdim), lambda i: (0, 0)),
                  pl.BlockSpec((h2w,), lambda i: (0,)),
                  pl.BlockSpec((h1w, h2w), lambda i: (0, 0)),
                  pl.BlockSpec((h1w,), lambda i: (0,)),
                  pl.BlockSpec((1, h1w), lambda i: (0, 0)),
                  pl.BlockSpec((1,), lambda i: (0,))],
        out_specs=pl.BlockSpec((r,), lambda i: (i,)),
        out_shape=jax.ShapeDtypeStruct((n,), jnp.float32),
    )(h1, h2, h3, w0a, w0b, w0c, b0, w1, b1, w2, b2)


def _tc_layer3_mlp(agg, scales, w, b, h1, h2,
                   w0a, w0b, w0c, b0, w1, b1, w2, b2, r=1024):
    n, hdim = agg.shape
    h2w = w0a.shape[0]
    h1w = w1.shape[0]

    def body(a_ref, s_ref, w_ref, b_ref, h1_ref, h2_ref,
             w0a_ref, w0b_ref, w0c_ref, b0_ref, w1_ref, b1_ref,
             w2_ref, b2_ref, o_ref):
        din_s = s_ref[1, :, 0:1]
        z3 = lax.dot_general(a_ref[...] * din_s, w_ref[...],
                             (((1,), (0,)), ((), ())),
                             preferred_element_type=jnp.float32)
        h3 = jnp.maximum(z3 + b_ref[...][None, :], 0.0)
        dn = (((1,), (1,)), ((), ()))
        z = (lax.dot_general(h1_ref[...], w0a_ref[...], dn,
                             preferred_element_type=jnp.float32)
             + lax.dot_general(h2_ref[...], w0b_ref[...], dn,
                               preferred_element_type=jnp.float32)
             + lax.dot_general(h3, w0c_ref[...], dn,
                               preferred_element_type=jnp.float32))
        a0 = jnp.maximum(z + b0_ref[...][None, :], 0.0)
        a1 = lax.dot_general(a0, w1_ref[...], dn,
                             preferred_element_type=jnp.float32)
        a1 = jnp.maximum(a1 + b1_ref[...][None, :], 0.0)
        logit = jnp.sum(a1 * w2_ref[...], axis=1) + b2_ref[0]
        o_ref[...] = 1.0 / (1.0 + jnp.exp(-logit))

    return pl.pallas_call(
        body,
        grid=(pl.cdiv(n, r),),
        in_specs=[pl.BlockSpec((r, hdim), lambda i: (i, 0)),
                  pl.BlockSpec((2, r, 16), lambda i: (0, i, 0)),
                  pl.BlockSpec((hdim, hdim), lambda i: (0, 0)),
                  pl.BlockSpec((hdim,), lambda i: (0,)),
                  pl.BlockSpec((r, hdim), lambda i: (i, 0)),
                  pl.BlockSpec((r, hdim), lambda i: (i, 0)),
                  pl.BlockSpec((h2w, hdim), lambda i: (0, 0)),
                  pl.BlockSpec((h2w, hdim), lambda i: (0, 0)),
                  pl.BlockSpec((h2w, hdim), lambda i: (0, 0)),
                  pl.BlockSpec((h2w,), lambda i: (0,)),
                  pl.BlockSpec((h1w, h2w), lambda i: (0, 0)),
                  pl.BlockSpec((h1w,), lambda i: (0,)),
                  pl.BlockSpec((1, h1w), lambda i: (0, 0)),
                  pl.BlockSpec((1,), lambda i: (0,))],
        out_specs=pl.BlockSpec((r,), lambda i: (i,)),
        out_shape=jax.ShapeDtypeStruct((n,), jnp.float32),
    )(agg, scales, w, b, h1, h2, w0a, w0b, w0c, b0, w1, b1, w2, b2)


def kernel(x, edge_index, edge_types, merge_W, merge_b,
           gcn_W0, gcn_b0, gcn_W1, gcn_b1, gcn_W2, gcn_b2,
           mlp_W0, mlp_b0, mlp_W1, mlp_b1, mlp_W2, mlp_b2):
    n, d = x.shape
    e = edge_index.shape[1]
    nch = e // (_NW * _C)
    src3 = edge_index[0].reshape(_NW, nch, _C)
    dst3 = edge_index[1].reshape(_NW, nch, _C)
    cq = 32
    nch2 = e // (_NS * cq)
    src2 = edge_index[0].reshape(_NS, nch2, cq)
    dst2 = edge_index[1].reshape(_NS, nch2, cq)
    ones16 = jnp.ones((_C, 16), jnp.float32)
    zeros16 = jnp.zeros((_ZR, 16), jnp.float32)
    zeros_rows = jnp.zeros((_ZR, d), jnp.float32)

    deg = _sc_degrees(src3, dst3, ones16, zeros16, n)
    hn, scales = _tc_merge(x, merge_W, merge_b, deg)

    hs = []
    for w, b in ((gcn_W0, gcn_b0), (gcn_W1, gcn_b1)):
        agg = _sc_msg(hn, src2, dst2, zeros_rows)
        h_l, hn = _tc_layer(agg, scales, w, b)
        hs.append(h_l)
    agg3 = _sc_msg(hn, src2, dst2, zeros_rows)

    hdim = d
    w0a = mlp_W0[:, :hdim]
    w0b = mlp_W0[:, hdim:2 * hdim]
    w0c = mlp_W0[:, 2 * hdim:]
    return _tc_layer3_mlp(agg3, scales, gcn_W2, gcn_b2, hs[0], hs[1],
                          w0a, w0b, w0c, mlp_b0, mlp_W1, mlp_b1,
                          mlp_W2, mlp_b2)


# speedup vs baseline: 1.0587x; 1.0227x over previous
import functools

import jax
import jax.numpy as jnp
from jax import lax
from jax.experimental import pallas as pl
from jax.experimental.pallas import tpu as pltpu
from jax.experimental.pallas import tpu_sc as plsc

_NC = 2
_NS = 16
_NW = _NC * _NS
_C = 80
_SLABS = 10
_ZR = 40


def _sc_mesh():
    return plsc.VectorSubcoreMesh(
        core_axis_name="c", subcore_axis_name="s",
        num_cores=_NC, num_subcores=_NS)


def _sc_degrees(src3, dst3, ones_hbm, zeros_hbm, n):
    _, nch, c = src3.shape
    slab = n // _SLABS

    @functools.partial(
        pl.kernel,
        out_type=jax.ShapeDtypeStruct((_NC, 2, n, 16), jnp.float32),
        mesh=_sc_mesh(),
        scratch_types=[
            pltpu.VMEM((nch, c), jnp.int32),
            pltpu.VMEM((nch, c), jnp.int32),
            pltpu.VMEM((c, 16), jnp.float32),
            pltpu.VMEM((_ZR, 16), jnp.float32),
            pltpu.VMEM_SHARED((n, 16), jnp.float32),
            pltpu.VMEM_SHARED((n, 16), jnp.float32),
            pltpu.SemaphoreType.DMA,
            pltpu.SemaphoreType.DMA,
        ],
        compiler_params=pltpu.CompilerParams(use_tc_tiling_on_sc=False),
    )
    def k(src_hbm, dst_hbm, ones_ref, zeros_ref, out_hbm,
          src_v, dst_v, ones_v, zb_v, dout_sh, din_sh, so, si):
        cid = lax.axis_index("c")
        sid = lax.axis_index("s")
        wid = sid * _NC + cid
        pltpu.sync_copy(ones_ref, ones_v)
        pltpu.sync_copy(zeros_ref, zb_v)

        @pl.when(sid < _SLABS)
        def _():
            for t in range(slab // _ZR):
                zsl = pl.ds(sid * slab + t * _ZR, _ZR)
                pltpu.sync_copy(zb_v, dout_sh.at[zsl])
                pltpu.sync_copy(zb_v, din_sh.at[zsl])

        pltpu.sync_copy(src_hbm.at[wid], src_v)
        pltpu.sync_copy(dst_hbm.at[wid], dst_v)
        plsc.subcore_barrier()

        win = 8

        def body(j, carry):
            pltpu.async_copy(ones_v, dout_sh.at[src_v.at[j]], so, add=True)
            pltpu.async_copy(ones_v, din_sh.at[dst_v.at[j]], si, add=True)

            @pl.when(j >= win)
            def _():
                pltpu.make_async_copy(ones_v, dout_sh.at[src_v.at[0]],
                                      so).wait()
                pltpu.make_async_copy(ones_v, din_sh.at[dst_v.at[0]],
                                      si).wait()

            return carry

        lax.fori_loop(0, nch, body, 0)
        for _ in range(win):
            pltpu.make_async_copy(ones_v, dout_sh.at[src_v.at[0]], so).wait()
            pltpu.make_async_copy(ones_v, din_sh.at[dst_v.at[0]], si).wait()
        plsc.subcore_barrier()

        @pl.when(sid < _SLABS)
        def _():
            sl = pl.ds(sid * slab, slab)
            for t in range(slab // _ZR):
                zsl = pl.ds(sid * slab + t * _ZR, _ZR)
                pltpu.sync_copy(dout_sh.at[zsl], zb_v)
                pltpu.sync_copy(zb_v, out_hbm.at[cid, 0, zsl])
                pltpu.sync_copy(din_sh.at[zsl], zb_v)
                pltpu.sync_copy(zb_v, out_hbm.at[cid, 1, zsl])

    return k(src3, dst3, ones_hbm, zeros_hbm)


def _sc_msg(hn, src2, dst2, zeros_hbm):
    n, d = hn.shape
    _, nch, c = src2.shape
    half = n // _NC
    garbage = half
    nslab = 5
    slab = half // nslab
    nbuf = 5

    @functools.partial(
        pl.kernel,
        out_type=jax.ShapeDtypeStruct((n, d), jnp.float32),
        mesh=_sc_mesh(),
        scratch_types=[
            pltpu.VMEM((nch, c), jnp.int32),
            pltpu.VMEM((nch, c), jnp.int32),
            pltpu.VMEM((nch, c), jnp.int32),
        ] + [pltpu.VMEM((c, d), jnp.float32)] * nbuf + [
            pltpu.VMEM((_ZR, d), jnp.float32),
            pltpu.VMEM((_ZR, d), jnp.float32),
            pltpu.VMEM_SHARED((half + 8, d), jnp.float32),
        ] + [pltpu.SemaphoreType.DMA] * (2 * nbuf),
        compiler_params=pltpu.CompilerParams(use_tc_tiling_on_sc=False),
    )
    def k(hn_hbm, src_hbm, dst_hbm, z_hbm, out_hbm,
          src_v, dst_v, dstm_v, *rest):
        bufs = rest[:nbuf]
        zb_v = rest[nbuf]
        zb2_v = rest[nbuf + 1]
        acc_sh = rest[nbuf + 2]
        gsem = rest[nbuf + 3:2 * nbuf + 3]
        ssem = rest[2 * nbuf + 3:]
        cid = lax.axis_index("c")
        sid = lax.axis_index("s")
        lo = cid * half
        pltpu.sync_copy(z_hbm, zb_v)
        pltpu.sync_copy(src_hbm.at[sid], src_v)
        pltpu.sync_copy(dst_hbm.at[sid], dst_v)

        @pl.when(sid < nslab)
        def _():
            for t in range(slab // _ZR):
                pltpu.async_copy(
                    zb_v, acc_sh.at[pl.ds(sid * slab + t * _ZR, _ZR)],
                    gsem[0])
            for t in range(slab // _ZR):
                pltpu.make_async_copy(
                    zb_v, acc_sh.at[pl.ds(sid * slab, _ZR)], gsem[0]).wait()

        def remap(r, carry):
            for cc in range(c // 16):
                dv = dst_v[r, pl.ds(cc * 16, 16)] - lo
                ok = (dv >= 0) & (dv < half)
                dstm_v[r, pl.ds(cc * 16, 16)] = jnp.where(ok, dv, garbage)
            return carry

        lax.fori_loop(0, nch, remap, 0)
        plsc.subcore_barrier()

        for b in range(nbuf):
            pltpu.async_copy(hn_hbm.at[src_v.at[b]], bufs[b], gsem[b])
        niter = nch // nbuf

        def body(i, carry):
            j = nbuf * i
            for b in range(nbuf):
                pltpu.make_async_copy(
                    hn_hbm.at[src_v.at[0]], bufs[b], gsem[b]).wait()
                pltpu.async_copy(bufs[b], acc_sh.at[dstm_v.at[j + b]],
                                 ssem[b], add=True)

            @pl.when(i < niter - 1)
            def _():
                for b in range(nbuf):
                    pltpu.make_async_copy(
                        bufs[b], acc_sh.at[dstm_v.at[0]], ssem[b]).wait()
                    pltpu.async_copy(hn_hbm.at[src_v.at[j + nbuf + b]],
                                     bufs[b], gsem[b])

            return carry

        lax.fori_loop(0, niter, body, 0)
        for b in range(nbuf):
            pltpu.make_async_copy(bufs[b], acc_sh.at[dstm_v.at[0]],
                                  ssem[b]).wait()
        plsc.subcore_barrier()

        @pl.when(sid < nslab)
        def _():
            wbufs = (zb_v, zb2_v)
            nt = slab // _ZR
            for t in range(nt):
                sl0 = sid * slab + t * _ZR
                wb = wbufs[t % 2]
                if t >= 2:
                    pltpu.make_async_copy(
                        wb, out_hbm.at[pl.ds(cid * half, _ZR)],
                        gsem[t % 2]).wait()
                pltpu.sync_copy(acc_sh.at[pl.ds(sl0, _ZR)], wb)
                pltpu.async_copy(
                    wb, out_hbm.at[pl.ds(cid * half + sl0, _ZR)],
                    gsem[t % 2])
            for t in (nt - 2, nt - 1):
                pltpu.make_async_copy(
                    wbufs[t % 2], out_hbm.at[pl.ds(cid * half, _ZR)],
                    gsem[t % 2]).wait()

    return k(hn, src2, dst2, zeros_hbm)


def _tc_merge(x, w, b, deg, r=1000):
    n, d = x.shape
    h = w.shape[0]

    def body(x_ref, w_ref, b_ref, deg_ref, hn_ref, sc_ref):
        dout_p = deg_ref[0, 0, :, 0:1] + deg_ref[1, 0, :, 0:1]
        din_p = deg_ref[0, 1, :, 0:1] + deg_ref[1, 1, :, 0:1]
        dout_s = lax.rsqrt(jnp.maximum(dout_p, 1.0))
        din_s = lax.rsqrt(jnp.maximum(din_p, 1.0))
        h0 = lax.dot_general(x_ref[...], w_ref[...], (((1,), (1,)), ((), ())),
                             preferred_element_type=jnp.float32)
        hn_ref[...] = (h0 + b_ref[...][None, :]) * dout_s
        sc_ref[0] = jnp.broadcast_to(dout_s, (r, 16))
        sc_ref[1] = jnp.broadcast_to(din_s, (r, 16))

    return pl.pallas_call(
        body,
        grid=(n // r,),
        in_specs=[pl.BlockSpec((r, d), lambda i: (i, 0)),
                  pl.BlockSpec((h, d), lambda i: (0, 0)),
                  pl.BlockSpec((h,), lambda i: (0,)),
                  pl.BlockSpec((2, 2, r, 16), lambda i: (0, 0, i, 0))],
        out_specs=[pl.BlockSpec((r, h), lambda i: (i, 0)),
                   pl.BlockSpec((2, r, 16), lambda i: (0, i, 0))],
        out_shape=[jax.ShapeDtypeStruct((n, h), jnp.float32),
                   jax.ShapeDtypeStruct((2, n, 16), jnp.float32)],
    )(x, w, b, deg)


def _tc_layer(agg, scales, w, b, r=1000):
    n, hdim = agg.shape

    def body(a_ref, s_ref, w_ref, b_ref, h_ref, hn_ref):
        dout_s = s_ref[0, :, 0:1]
        din_s = s_ref[1, :, 0:1]
        z = lax.dot_general(a_ref[...] * din_s, w_ref[...],
                            (((1,), (0,)), ((), ())),
                            preferred_element_type=jnp.float32)
        hl = jnp.maximum(z + b_ref[...][None, :], 0.0)
        h_ref[...] = hl
        hn_ref[...] = hl * dout_s

    return pl.pallas_call(
        body,
        grid=(n // r,),
        in_specs=[pl.BlockSpec((r, hdim), lambda i: (i, 0)),
                  pl.BlockSpec((2, r, 16), lambda i: (0, i, 0)),
                  pl.BlockSpec((hdim, hdim), lambda i: (0, 0)),
                  pl.BlockSpec((hdim,), lambda i: (0,))],
        out_specs=[pl.BlockSpec((r, hdim), lambda i: (i, 0)),
                   pl.BlockSpec((r, hdim), lambda i: (i, 0))],
        out_shape=[jax.ShapeDtypeStruct((n, hdim), jnp.float32),
                   jax.ShapeDtypeStruct((n, hdim), jnp.float32)],
    )(agg, scales, w, b)


def _tc_mlp(h1, h2, h3, w0a, w0b, w0c, b0, w1, b1, w2, b2, r=1024):
    n, hdim = h1.shape
    h2w = w0a.shape[0]
    h1w = w1.shape[0]

    def body(h1_ref, h2_ref, h3_ref, w0a_ref, w0b_ref, w0c_ref, b0_ref,
             w1_ref, b1_ref, w2_ref, b2_ref, o_ref):
        dn = (((1,), (1,)), ((), ()))
        z = (lax.dot_general(h1_ref[...], w0a_ref[...], dn,
                             preferred_element_type=jnp.float32)
             + lax.dot_general(h2_ref[...], w0b_ref[...], dn,
                               preferred_element_type=jnp.float32)
             + lax.dot_general(h3_ref[...], w0c_ref[...], dn,
                               preferred_element_type=jnp.float32))
        a0 = jnp.maximum(z + b0_ref[...][None, :], 0.0)
        a1 = lax.dot_general(a0, w1_ref[...], dn,
                             preferred_element_type=jnp.float32)
        a1 = jnp.maximum(a1 + b1_ref[...][None, :], 0.0)
        logit = jnp.sum(a1 * w2_ref[...], axis=1) + b2_ref[0]
        o_ref[...] = 1.0 / (1.0 + jnp.exp(-logit))

    return pl.pallas_call(
        body,
        grid=(pl.cdiv(n, r),),
        in_specs=[pl.BlockSpec((r, hdim), lambda i: (i, 0)),
                  pl.BlockSpec((r, hdim), lambda i: (i, 0)),
                  pl.BlockSpec((r, hdim), lambda i: (i, 0)),
                  pl.BlockSpec((h2w, hdim), lambda i: (0, 0)),
                  pl.BlockSpec((h2w, hdim), lambda i: (0, 0)),
                  pl.BlockSpec((h2w, hdim), lambda i: (0, 0)),
                  pl.BlockSpec((h2w,), lambda i: (0,)),
                  pl.BlockSpec((h1w, h2w), lambda i: (0, 0)),
                  pl.BlockSpec((h1w,), lambda i: (0,)),
                  pl.BlockSpec((1, h1w), lambda i: (0, 0)),
                  pl.BlockSpec((1,), lambda i: (0,))],
        out_specs=pl.BlockSpec((r,), lambda i: (i,)),
        out_shape=jax.ShapeDtypeStruct((n,), jnp.float32),
    )(h1, h2, h3, w0a, w0b, w0c, b0, w1, b1, w2, b2)


def _tc_layer3_mlp(agg, scales, w, b, h1, h2,
                   w0a, w0b, w0c, b0, w1, b1, w2, b2, r=1024):
    n, hdim = agg.shape
    h2w = w0a.shape[0]
    h1w = w1.shape[0]

    def body(a_ref, s_ref, w_ref, b_ref, h1_ref, h2_ref,
             w0a_ref, w0b_ref, w0c_ref, b0_ref, w1_ref, b1_ref,
             w2_ref, b2_ref, o_ref):
        din_s = s_ref[1, :, 0:1]
        z3 = lax.dot_general(a_ref[...] * din_s, w_ref[...],
                             (((1,), (0,)), ((), ())),
                             preferred_element_type=jnp.float32)
        h3 = jnp.maximum(z3 + b_ref[...][None, :], 0.0)
        dn = (((1,), (1,)), ((), ()))
        z = (lax.dot_general(h1_ref[...], w0a_ref[...], dn,
                             preferred_element_type=jnp.float32)
             + lax.dot_general(h2_ref[...], w0b_ref[...], dn,
                               preferred_element_type=jnp.float32)
             + lax.dot_general(h3, w0c_ref[...], dn,
                               preferred_element_type=jnp.float32))
        a0 = jnp.maximum(z + b0_ref[...][None, :], 0.0)
        a1 = lax.dot_general(a0, w1_ref[...], dn,
                             preferred_element_type=jnp.float32)
        a1 = jnp.maximum(a1 + b1_ref[...][None, :], 0.0)
        logit = jnp.sum(a1 * w2_ref[...], axis=1) + b2_ref[0]
        o_ref[...] = 1.0 / (1.0 + jnp.exp(-logit))

    return pl.pallas_call(
        body,
        grid=(pl.cdiv(n, r),),
        in_specs=[pl.BlockSpec((r, hdim), lambda i: (i, 0)),
                  pl.BlockSpec((2, r, 16), lambda i: (0, i, 0)),
                  pl.BlockSpec((hdim, hdim), lambda i: (0, 0)),
                  pl.BlockSpec((hdim,), lambda i: (0,)),
                  pl.BlockSpec((r, hdim), lambda i: (i, 0)),
                  pl.BlockSpec((r, hdim), lambda i: (i, 0)),
                  pl.BlockSpec((h2w, hdim), lambda i: (0, 0)),
                  pl.BlockSpec((h2w, hdim), lambda i: (0, 0)),
                  pl.BlockSpec((h2w, hdim), lambda i: (0, 0)),
                  pl.BlockSpec((h2w,), lambda i: (0,)),
                  pl.BlockSpec((h1w, h2w), lambda i: (0, 0)),
                  pl.BlockSpec((h1w,), lambda i: (0,)),
                  pl.BlockSpec((1, h1w), lambda i: (0, 0)),
                  pl.BlockSpec((1,), lambda i: (0,))],
        out_specs=pl.BlockSpec((r,), lambda i: (i,)),
        out_shape=jax.ShapeDtypeStruct((n,), jnp.float32),
    )(agg, scales, w, b, h1, h2, w0a, w0b, w0c, b0, w1, b1, w2, b2)


def kernel(x, edge_index, edge_types, merge_W, merge_b,
           gcn_W0, gcn_b0, gcn_W1, gcn_b1, gcn_W2, gcn_b2,
           mlp_W0, mlp_b0, mlp_W1, mlp_b1, mlp_W2, mlp_b2):
    n, d = x.shape
    e = edge_index.shape[1]
    nch = e // (_NW * _C)
    src3 = edge_index[0].reshape(_NW, nch, _C)
    dst3 = edge_index[1].reshape(_NW, nch, _C)
    cq = 32
    nch2 = e // (_NS * cq)
    src2 = edge_index[0].reshape(_NS, nch2, cq)
    dst2 = edge_index[1].reshape(_NS, nch2, cq)
    ones16 = jnp.ones((_C, 16), jnp.float32)
    zeros16 = jnp.zeros((_ZR, 16), jnp.float32)
    zeros_rows = jnp.zeros((_ZR, d), jnp.float32)

    deg = _sc_degrees(src3, dst3, ones16, zeros16, n)
    hn, scales = _tc_merge(x, merge_W, merge_b, deg)

    hs = []
    for w, b in ((gcn_W0, gcn_b0), (gcn_W1, gcn_b1)):
        agg = _sc_msg(hn, src2, dst2, zeros_rows)
        h_l, hn = _tc_layer(agg, scales, w, b)
        hs.append(h_l)
    agg3 = _sc_msg(hn, src2, dst2, zeros_rows)

    hdim = d
    w0a = mlp_W0[:, :hdim]
    w0b = mlp_W0[:, hdim:2 * hdim]
    w0c = mlp_W0[:, 2 * hdim:]
    return _tc_layer3_mlp(agg3, scales, gcn_W2, gcn_b2, hs[0], hs[1],
                          w0a, w0b, w0c, mlp_b0, mlp_W1, mlp_b1,
                          mlp_W2, mlp_b2)


# speedup vs baseline: 1.0642x; 1.0052x over previous
import functools

import jax
import jax.numpy as jnp
from jax import lax
from jax.experimental import pallas as pl
from jax.experimental.pallas import tpu as pltpu
from jax.experimental.pallas import tpu_sc as plsc

_NC = 2
_NS = 16
_NW = _NC * _NS
_C = 80
_SLABS = 10
_ZR = 40


def _sc_mesh():
    return plsc.VectorSubcoreMesh(
        core_axis_name="c", subcore_axis_name="s",
        num_cores=_NC, num_subcores=_NS)


def _sc_degrees(src3, dst3, ones_hbm, zeros_hbm, n):
    _, nch, c = src3.shape
    slab = n // _SLABS

    @functools.partial(
        pl.kernel,
        out_type=jax.ShapeDtypeStruct((_NC, 2, n, 16), jnp.float32),
        mesh=_sc_mesh(),
        scratch_types=[
            pltpu.VMEM((nch, c), jnp.int32),
            pltpu.VMEM((nch, c), jnp.int32),
            pltpu.VMEM((c, 16), jnp.float32),
            pltpu.VMEM((_ZR, 16), jnp.float32),
            pltpu.VMEM((_ZR, 16), jnp.float32),
            pltpu.VMEM_SHARED((n, 16), jnp.float32),
            pltpu.VMEM_SHARED((n, 16), jnp.float32),
            pltpu.SemaphoreType.DMA,
            pltpu.SemaphoreType.DMA,
        ],
        compiler_params=pltpu.CompilerParams(use_tc_tiling_on_sc=False),
    )
    def k(src_hbm, dst_hbm, ones_ref, zeros_ref, out_hbm,
          src_v, dst_v, ones_v, zb_v, zb2_v, dout_sh, din_sh, so, si):
        cid = lax.axis_index("c")
        sid = lax.axis_index("s")
        wid = sid * _NC + cid
        pltpu.sync_copy(ones_ref, ones_v)
        pltpu.sync_copy(zeros_ref, zb_v)

        @pl.when(sid < _SLABS)
        def _():
            for t in range(slab // _ZR):
                zsl = pl.ds(sid * slab + t * _ZR, _ZR)
                pltpu.async_copy(zb_v, dout_sh.at[zsl], so)
                pltpu.async_copy(zb_v, din_sh.at[zsl], si)
            for t in range(slab // _ZR):
                zs0 = pl.ds(sid * slab, _ZR)
                pltpu.make_async_copy(zb_v, dout_sh.at[zs0], so).wait()
                pltpu.make_async_copy(zb_v, din_sh.at[zs0], si).wait()

        pltpu.sync_copy(src_hbm.at[wid], src_v)
        pltpu.sync_copy(dst_hbm.at[wid], dst_v)
        plsc.subcore_barrier()

        win = 8

        def body(j, carry):
            pltpu.async_copy(ones_v, dout_sh.at[src_v.at[j]], so, add=True)
            pltpu.async_copy(ones_v, din_sh.at[dst_v.at[j]], si, add=True)

            @pl.when(j >= win)
            def _():
                pltpu.make_async_copy(ones_v, dout_sh.at[src_v.at[0]],
                                      so).wait()
                pltpu.make_async_copy(ones_v, din_sh.at[dst_v.at[0]],
                                      si).wait()

            return carry

        lax.fori_loop(0, nch, body, 0)
        for _ in range(win):
            pltpu.make_async_copy(ones_v, dout_sh.at[src_v.at[0]], so).wait()
            pltpu.make_async_copy(ones_v, din_sh.at[dst_v.at[0]], si).wait()
        plsc.subcore_barrier()

        @pl.when(sid < _SLABS)
        def _():
            nt = slab // _ZR
            for t in range(nt):
                zsl = pl.ds(sid * slab + t * _ZR, _ZR)
                zs0 = pl.ds(sid * slab, _ZR)
                wb = zb_v if t % 2 == 0 else zb2_v
                sem = so if t % 2 == 0 else si
                if t >= 2:
                    pltpu.make_async_copy(wb, out_hbm.at[cid, 1, zs0],
                                          sem).wait()
                pltpu.sync_copy(dout_sh.at[zsl], wb)
                pltpu.async_copy(wb, out_hbm.at[cid, 0, zsl], sem)
                pltpu.make_async_copy(wb, out_hbm.at[cid, 0, zs0], sem).wait()
                pltpu.sync_copy(din_sh.at[zsl], wb)
                pltpu.async_copy(wb, out_hbm.at[cid, 1, zsl], sem)
            for t in (nt - 2, nt - 1):
                wb = zb_v if t % 2 == 0 else zb2_v
                sem = so if t % 2 == 0 else si
                pltpu.make_async_copy(wb, out_hbm.at[cid, 1,
                                                     pl.ds(sid * slab, _ZR)],
                                      sem).wait()

    return k(src3, dst3, ones_hbm, zeros_hbm)


def _sc_msg(hn, src2, dst2, zeros_hbm):
    n, d = hn.shape
    _, nch, c = src2.shape
    half = n // _NC
    garbage = half
    nslab = 5
    slab = half // nslab
    nbuf = 5

    @functools.partial(
        pl.kernel,
        out_type=jax.ShapeDtypeStruct((n, d), jnp.float32),
        mesh=_sc_mesh(),
        scratch_types=[
            pltpu.VMEM((nch, c), jnp.int32),
            pltpu.VMEM((nch, c), jnp.int32),
            pltpu.VMEM((nch, c), jnp.int32),
        ] + [pltpu.VMEM((c, d), jnp.float32)] * nbuf + [
            pltpu.VMEM((_ZR, d), jnp.float32),
            pltpu.VMEM((_ZR, d), jnp.float32),
            pltpu.VMEM_SHARED((half + 8, d), jnp.float32),
        ] + [pltpu.SemaphoreType.DMA] * (2 * nbuf),
        compiler_params=pltpu.CompilerParams(use_tc_tiling_on_sc=False),
    )
    def k(hn_hbm, src_hbm, dst_hbm, z_hbm, out_hbm,
          src_v, dst_v, dstm_v, *rest):
        bufs = rest[:nbuf]
        zb_v = rest[nbuf]
        zb2_v = rest[nbuf + 1]
        acc_sh = rest[nbuf + 2]
        gsem = rest[nbuf + 3:2 * nbuf + 3]
        ssem = rest[2 * nbuf + 3:]
        cid = lax.axis_index("c")
        sid = lax.axis_index("s")
        lo = cid * half
        pltpu.sync_copy(z_hbm, zb_v)
        pltpu.sync_copy(src_hbm.at[sid], src_v)
        pltpu.sync_copy(dst_hbm.at[sid], dst_v)

        @pl.when(sid < nslab)
        def _():
            for t in range(slab // _ZR):
                pltpu.async_copy(
                    zb_v, acc_sh.at[pl.ds(sid * slab + t * _ZR, _ZR)],
                    gsem[0])
            for t in range(slab // _ZR):
                pltpu.make_async_copy(
                    zb_v, acc_sh.at[pl.ds(sid * slab, _ZR)], gsem[0]).wait()

        def remap(r, carry):
            for cc in range(c // 16):
                dv = dst_v[r, pl.ds(cc * 16, 16)] - lo
                ok = (dv >= 0) & (dv < half)
                dstm_v[r, pl.ds(cc * 16, 16)] = jnp.where(ok, dv, garbage)
            return carry

        lax.fori_loop(0, nch, remap, 0)
        plsc.subcore_barrier()

        for b in range(nbuf):
            pltpu.async_copy(hn_hbm.at[src_v.at[b]], bufs[b], gsem[b])
        niter = nch // nbuf

        def body(i, carry):
            j = nbuf * i
            for b in range(nbuf):
                pltpu.make_async_copy(
                    hn_hbm.at[src_v.at[0]], bufs[b], gsem[b]).wait()
                pltpu.async_copy(bufs[b], acc_sh.at[dstm_v.at[j + b]],
                                 ssem[b], add=True)

            @pl.when(i < niter - 1)
            def _():
                for b in range(nbuf):
                    pltpu.make_async_copy(
                        bufs[b], acc_sh.at[dstm_v.at[0]], ssem[b]).wait()
                    pltpu.async_copy(hn_hbm.at[src_v.at[j + nbuf + b]],
                                     bufs[b], gsem[b])

            return carry

        lax.fori_loop(0, niter, body, 0)
        for b in range(nbuf):
            pltpu.make_async_copy(bufs[b], acc_sh.at[dstm_v.at[0]],
                                  ssem[b]).wait()
        plsc.subcore_barrier()

        @pl.when(sid < nslab)
        def _():
            wbufs = (zb_v, zb2_v)
            nt = slab // _ZR
            for t in range(nt):
                sl0 = sid * slab + t * _ZR
                wb = wbufs[t % 2]
                if t >= 2:
                    pltpu.make_async_copy(
                        wb, out_hbm.at[pl.ds(cid * half, _ZR)],
                        gsem[t % 2]).wait()
                pltpu.sync_copy(acc_sh.at[pl.ds(sl0, _ZR)], wb)
                pltpu.async_copy(
                    wb, out_hbm.at[pl.ds(cid * half + sl0, _ZR)],
                    gsem[t % 2])
            for t in (nt - 2, nt - 1):
                pltpu.make_async_copy(
                    wbufs[t % 2], out_hbm.at[pl.ds(cid * half, _ZR)],
                    gsem[t % 2]).wait()

    return k(hn, src2, dst2, zeros_hbm)


def _tc_merge(x, w, b, deg, r=1000):
    n, d = x.shape
    h = w.shape[0]

    def body(x_ref, w_ref, b_ref, deg_ref, hn_ref, sc_ref):
        dout_p = deg_ref[0, 0, :, 0:1] + deg_ref[1, 0, :, 0:1]
        din_p = deg_ref[0, 1, :, 0:1] + deg_ref[1, 1, :, 0:1]
        dout_s = lax.rsqrt(jnp.maximum(dout_p, 1.0))
        din_s = lax.rsqrt(jnp.maximum(din_p, 1.0))
        h0 = lax.dot_general(x_ref[...], w_ref[...], (((1,), (1,)), ((), ())),
                             preferred_element_type=jnp.float32)
        hn_ref[...] = (h0 + b_ref[...][None, :]) * dout_s
        sc_ref[0] = jnp.broadcast_to(dout_s, (r, 16))
        sc_ref[1] = jnp.broadcast_to(din_s, (r, 16))

    return pl.pallas_call(
        body,
        grid=(n // r,),
        in_specs=[pl.BlockSpec((r, d), lambda i: (i, 0)),
                  pl.BlockSpec((h, d), lambda i: (0, 0)),
                  pl.BlockSpec((h,), lambda i: (0,)),
                  pl.BlockSpec((2, 2, r, 16), lambda i: (0, 0, i, 0))],
        out_specs=[pl.BlockSpec((r, h), lambda i: (i, 0)),
                   pl.BlockSpec((2, r, 16), lambda i: (0, i, 0))],
        out_shape=[jax.ShapeDtypeStruct((n, h), jnp.float32),
                   jax.ShapeDtypeStruct((2, n, 16), jnp.float32)],
    )(x, w, b, deg)


def _tc_layer(agg, scales, w, b, r=1000):
    n, hdim = agg.shape

    def body(a_ref, s_ref, w_ref, b_ref, h_ref, hn_ref):
        dout_s = s_ref[0, :, 0:1]
        din_s = s_ref[1, :, 0:1]
        z = lax.dot_general(a_ref[...] * din_s, w_ref[...],
                            (((1,), (0,)), ((), ())),
                            preferred_element_type=jnp.float32)
        hl = jnp.maximum(z + b_ref[...][None, :], 0.0)
        h_ref[...] = hl
        hn_ref[...] = hl * dout_s

    return pl.pallas_call(
        body,
        grid=(n // r,),
        in_specs=[pl.BlockSpec((r, hdim), lambda i: (i, 0)),
                  pl.BlockSpec((2, r, 16), lambda i: (0, i, 0)),
                  pl.BlockSpec((hdim, hdim), lambda i: (0, 0)),
                  pl.BlockSpec((hdim,), lambda i: (0,))],
        out_specs=[pl.BlockSpec((r, hdim), lambda i: (i, 0)),
                   pl.BlockSpec((r, hdim), lambda i: (i, 0))],
        out_shape=[jax.ShapeDtypeStruct((n, hdim), jnp.float32),
                   jax.ShapeDtypeStruct((n, hdim), jnp.float32)],
    )(agg, scales, w, b)


def _tc_mlp(h1, h2, h3, w0a, w0b, w0c, b0, w1, b1, w2, b2, r=1024):
    n, hdim = h1.shape
    h2w = w0a.shape[0]
    h1w = w1.shape[0]

    def body(h1_ref, h2_ref, h3_ref, w0a_ref, w0b_ref, w0c_ref, b0_ref,
             w1_ref, b1_ref, w2_ref, b2_ref, o_ref):
        dn = (((1,), (1,)), ((), ()))
        z = (lax.dot_general(h1_ref[...], w0a_ref[...], dn,
                             preferred_element_type=jnp.float32)
             + lax.dot_general(h2_ref[...], w0b_ref[...], dn,
                               preferred_element_type=jnp.float32)
             + lax.dot_general(h3_ref[...], w0c_ref[...], dn,
                               preferred_element_type=jnp.float32))
        a0 = jnp.maximum(z + b0_ref[...][None, :], 0.0)
        a1 = lax.dot_general(a0, w1_ref[...], dn,
                             preferred_element_type=jnp.float32)
        a1 = jnp.maximum(a1 + b1_ref[...][None, :], 0.0)
        logit = jnp.sum(a1 * w2_ref[...], axis=1) + b2_ref[0]
        o_ref[...] = 1.0 / (1.0 + jnp.exp(-logit))

    return pl.pallas_call(
        body,
        grid=(pl.cdiv(n, r),),
        in_specs=[pl.BlockSpec((r, hdim), lambda i: (i, 0)),
                  pl.BlockSpec((r, hdim), lambda i: (i, 0)),
                  pl.BlockSpec((r, hdim), lambda i: (i, 0)),
                  pl.BlockSpec((h2w, hdim), lambda i: (0, 0)),
                  pl.BlockSpec((h2w, hdim), lambda i: (0, 0)),
                  pl.BlockSpec((h2w, hdim), lambda i: (0, 0)),
                  pl.BlockSpec((h2w,), lambda i: (0,)),
                  pl.BlockSpec((h1w, h2w), lambda i: (0, 0)),
                  pl.BlockSpec((h1w,), lambda i: (0,)),
                  pl.BlockSpec((1, h1w), lambda i: (0, 0)),
                  pl.BlockSpec((1,), lambda i: (0,))],
        out_specs=pl.BlockSpec((r,), lambda i: (i,)),
        out_shape=jax.ShapeDtypeStruct((n,), jnp.float32),
    )(h1, h2, h3, w0a, w0b, w0c, b0, w1, b1, w2, b2)


def _tc_layer3_mlp(agg, scales, w, b, h1, h2,
                   w0a, w0b, w0c, b0, w1, b1, w2, b2, r=1024):
    n, hdim = agg.shape
    h2w = w0a.shape[0]
    h1w = w1.shape[0]

    def body(a_ref, s_ref, w_ref, b_ref, h1_ref, h2_ref,
             w0a_ref, w0b_ref, w0c_ref, b0_ref, w1_ref, b1_ref,
             w2_ref, b2_ref, o_ref):
        din_s = s_ref[1, :, 0:1]
        z3 = lax.dot_general(a_ref[...] * din_s, w_ref[...],
                             (((1,), (0,)), ((), ())),
                             preferred_element_type=jnp.float32)
        h3 = jnp.maximum(z3 + b_ref[...][None, :], 0.0)
        dn = (((1,), (1,)), ((), ()))
        z = (lax.dot_general(h1_ref[...], w0a_ref[...], dn,
                             preferred_element_type=jnp.float32)
             + lax.dot_general(h2_ref[...], w0b_ref[...], dn,
                               preferred_element_type=jnp.float32)
             + lax.dot_general(h3, w0c_ref[...], dn,
                               preferred_element_type=jnp.float32))
        a0 = jnp.maximum(z + b0_ref[...][None, :], 0.0)
        a1 = lax.dot_general(a0, w1_ref[...], dn,
                             preferred_element_type=jnp.float32)
        a1 = jnp.maximum(a1 + b1_ref[...][None, :], 0.0)
        logit = jnp.sum(a1 * w2_ref[...], axis=1) + b2_ref[0]
        o_ref[...] = 1.0 / (1.0 + jnp.exp(-logit))

    return pl.pallas_call(
        body,
        grid=(pl.cdiv(n, r),),
        in_specs=[pl.BlockSpec((r, hdim), lambda i: (i, 0)),
                  pl.BlockSpec((2, r, 16), lambda i: (0, i, 0)),
                  pl.BlockSpec((hdim, hdim), lambda i: (0, 0)),
                  pl.BlockSpec((hdim,), lambda i: (0,)),
                  pl.BlockSpec((r, hdim), lambda i: (i, 0)),
                  pl.BlockSpec((r, hdim), lambda i: (i, 0)),
                  pl.BlockSpec((h2w, hdim), lambda i: (0, 0)),
                  pl.BlockSpec((h2w, hdim), lambda i: (0, 0)),
                  pl.BlockSpec((h2w, hdim), lambda i: (0, 0)),
                  pl.BlockSpec((h2w,), lambda i: (0,)),
                  pl.BlockSpec((h1w, h2w), lambda i: (0, 0)),
                  pl.BlockSpec((h1w,), lambda i: (0,)),
                  pl.BlockSpec((1, h1w), lambda i: (0, 0)),
                  pl.BlockSpec((1,), lambda i: (0,))],
        out_specs=pl.BlockSpec((r,), lambda i: (i,)),
        out_shape=jax.ShapeDtypeStruct((n,), jnp.float32),
    )(agg, scales, w, b, h1, h2, w0a, w0b, w0c, b0, w1, b1, w2, b2)


def kernel(x, edge_index, edge_types, merge_W, merge_b,
           gcn_W0, gcn_b0, gcn_W1, gcn_b1, gcn_W2, gcn_b2,
           mlp_W0, mlp_b0, mlp_W1, mlp_b1, mlp_W2, mlp_b2):
    n, d = x.shape
    e = edge_index.shape[1]
    nch = e // (_NW * _C)
    src3 = edge_index[0].reshape(_NW, nch, _C)
    dst3 = edge_index[1].reshape(_NW, nch, _C)
    cq = 32
    nch2 = e // (_NS * cq)
    src2 = edge_index[0].reshape(_NS, nch2, cq)
    dst2 = edge_index[1].reshape(_NS, nch2, cq)
    ones16 = jnp.ones((_C, 16), jnp.float32)
    zeros16 = jnp.zeros((_ZR, 16), jnp.float32)
    zeros_rows = jnp.zeros((_ZR, d), jnp.float32)

    deg = _sc_degrees(src3, dst3, ones16, zeros16, n)
    hn, scales = _tc_merge(x, merge_W, merge_b, deg)

    hs = []
    for w, b in ((gcn_W0, gcn_b0), (gcn_W1, gcn_b1)):
        agg = _sc_msg(hn, src2, dst2, zeros_rows)
        h_l, hn = _tc_layer(agg, scales, w, b)
        hs.append(h_l)
    agg3 = _sc_msg(hn, src2, dst2, zeros_rows)

    hdim = d
    w0a = mlp_W0[:, :hdim]
    w0b = mlp_W0[:, hdim:2 * hdim]
    w0c = mlp_W0[:, 2 * hdim:]
    return _tc_layer3_mlp(agg3, scales, gcn_W2, gcn_b2, hs[0], hs[1],
                          w0a, w0b, w0c, mlp_b0, mlp_W1, mlp_b1,
                          mlp_W2, mlp_b2)
